# baseline (speedup 1.0000x reference)
import functools
import numpy as np

import jax
import jax.numpy as jnp
from jax import lax
from jax.experimental import pallas as pl
from jax.experimental.pallas import tpu as pltpu

F32 = jnp.float32
BF16 = jnp.bfloat16
LN_EPS = 1e-6
KSIZE = 7
HALO = 3
PAD_ROWS = 8
TOP = 8
_GELU_C = 0.7978845608028654


def _layer_norm(x, g, b):
    mu = jnp.mean(x, axis=-1, keepdims=True)
    var = jnp.mean((x - mu) ** 2, axis=-1, keepdims=True)
    return (x - mu) * lax.rsqrt(var + LN_EPS) * g + b


def _normalize(x):
    mu = jnp.mean(x, axis=-1, keepdims=True)
    var = jnp.mean((x - mu) ** 2, axis=-1, keepdims=True)
    return (x - mu) * lax.rsqrt(var + LN_EPS)


_GELU_C3 = _GELU_C * 0.044715


def _gelu_tanh(x):
    t = jnp.tanh(x * (_GELU_C + _GELU_C3 * (x * x)))
    half = 0.5 * x
    return half + half * t


def _fused_kernel(x_ref, ew_ref, eb_ref, ng_ref, nb_ref,
                  dww_ref, dwb_ref,
                  w1_ref, b1_ref, w2_ref, b2_ref,
                  wm_ref, bm_ref, wp_ref, bp_ref,
                  cosb_ref, sinb_ref, env_ref, o_ref,
                  xpad_ref, *, T, n_layers, hop):
    srows, dim = xpad_ref.shape
    xpad_ref[0:TOP, :] = jnp.zeros((TOP, dim), F32)
    xpad_ref[TOP + T:srows, :] = jnp.zeros((srows - TOP - T, dim), F32)

    acc = eb_ref[...]
    for j in range(KSIZE):
        acc = acc + jnp.dot(x_ref[0, j:j + T, :], ew_ref[j],
                            preferred_element_type=F32)
    xpad_ref[TOP:TOP + T, :] = _layer_norm(acc, ng_ref[...], nb_ref[...])

    for l in range(n_layers):
        y = dwb_ref[l]
        for j in range(KSIZE):
            y = y + xpad_ref[TOP - HALO + j:TOP - HALO + j + T, :] * dww_ref[l, j:j + 1, :]
        y = _normalize(y)
        h = jnp.dot(y, w1_ref[l], preferred_element_type=F32) + b1_ref[l]
        h = _gelu_tanh(h)
        h = jnp.dot(h, w2_ref[l], preferred_element_type=F32) + b2_ref[l]
        xpad_ref[TOP:TOP + T, :] += h

    xn = _normalize(xpad_ref[TOP:TOP + T, :])
    sm = jnp.dot(xn, wm_ref[...], preferred_element_type=F32) + bm_ref[...]
    sp = jnp.dot(xn, wp_ref[...], preferred_element_type=F32) + bp_ref[...]
    mag = jnp.minimum(jnp.exp(sm), 100.0)
    real = mag * jnp.cos(sp)
    imag = mag * jnp.sin(sp)
    frames = (jnp.dot(real, cosb_ref[...], preferred_element_type=F32)
              + jnp.dot(imag, sinb_ref[...], preferred_element_type=F32))

    r = frames.shape[1] // hop
    orows = o_ref.shape[1]
    ola = None
    for c in range(r):
        part = jnp.pad(frames[:, c * hop:(c + 1) * hop],
                       ((c, orows - T - c), (0, 0)))
        ola = part if ola is None else ola + part
    o_ref[0] = ola * env_ref[...]


def _inv_envelope_2d(t_valid, n_fft, hop, rows):
    n = np.arange(n_fft)
    w2 = (0.5 - 0.5 * np.cos(2.0 * np.pi * n / n_fft)) ** 2
    total = n_fft + (t_valid - 1) * hop
    env = np.zeros((total,), np.float64)
    for t in range(t_valid):
        env[t * hop:t * hop + n_fft] += w2
    start = n_fft // 2
    length = (t_valid - 1) * hop
    inv = 1.0 / np.maximum(env[start:start + length], 1e-8)
    full = np.zeros((rows * hop,), np.float64)
    full[start:start + length] = inv
    return jnp.asarray(full.reshape(rows, hop), F32)


def kernel(mel, embed_w, embed_b, norm_g, norm_b, final_g, final_b, out_wm, out_wp, out_bm, out_bp, cosb, sinb, dww_0, dwb_0, g_0, b_0, w1_0, b1_0, w2_0, b2_0, gamma_0, dww_1, dwb_1, g_1, b_1, w1_1, b1_1, w2_1, b2_1, gamma_1, dww_2, dwb_2, g_2, b_2, w1_2, b1_2, w2_2, b2_2, gamma_2, dww_3, dwb_3, g_3, b_3, w1_3, b1_3, w2_3, b2_3, gamma_3, dww_4, dwb_4, g_4, b_4, w1_4, b1_4, w2_4, b2_4, gamma_4, dww_5, dwb_5, g_5, b_5, w1_5, b1_5, w2_5, b2_5, gamma_5, dww_6, dwb_6, g_6, b_6, w1_6, b1_6, w2_6, b2_6, gamma_6, dww_7, dwb_7, g_7, b_7, w1_7, b1_7, w2_7, b2_7, gamma_7):
    B, n_mels, T = mel.shape
    dim = embed_w.shape[2]
    n_fft = cosb.shape[1]
    hop = n_fft // 4
    n_layers = 8
    rows = T + PAD_ROWS
    nm_pad = 128

    x = jnp.transpose(mel, (0, 2, 1))
    x = jnp.pad(x, ((0, 0), (HALO, PAD_ROWS - HALO), (0, nm_pad - n_mels)))
    ew = jnp.pad(embed_w, ((0, 0), (0, nm_pad - n_mels), (0, 0)))

    dww = jnp.stack([dww_0, dww_1, dww_2, dww_3, dww_4, dww_5, dww_6, dww_7])
    dwb = jnp.stack([dwb_0, dwb_1, dwb_2, dwb_3, dwb_4, dwb_5, dwb_6, dwb_7])

    gs = [g_0, g_1, g_2, g_3, g_4, g_5, g_6, g_7]
    bs = [b_0, b_1, b_2, b_3, b_4, b_5, b_6, b_7]
    w1s = [w1_0, w1_1, w1_2, w1_3, w1_4, w1_5, w1_6, w1_7]
    b1s = [b1_0, b1_1, b1_2, b1_3, b1_4, b1_5, b1_6, b1_7]
    w2s = [w2_0, w2_1, w2_2, w2_3, w2_4, w2_5, w2_6, w2_7]
    b2s = [b2_0, b2_1, b2_2, b2_3, b2_4, b2_5, b2_6, b2_7]
    gms = [gamma_0, gamma_1, gamma_2, gamma_3, gamma_4, gamma_5, gamma_6, gamma_7]
    w1 = jnp.stack([(gs[l][0, :, None] * w1s[l].astype(F32)).astype(BF16)
                    for l in range(n_layers)])
    b1 = jnp.stack([b1s[l] + jnp.dot(bs[l], w1s[l].astype(F32))
                    for l in range(n_layers)])
    w2 = jnp.stack([(w2s[l].astype(F32) * gms[l][0, None, :]).astype(BF16)
                    for l in range(n_layers)])
    b2 = jnp.stack([gms[l] * b2s[l] for l in range(n_layers)])

    wm = (final_g[0, :, None] * out_wm.astype(F32)).astype(BF16)
    bm = out_bm + jnp.dot(final_b, out_wm.astype(F32))
    wp = (final_g[0, :, None] * out_wp.astype(F32)).astype(BF16)
    bp = out_bp + jnp.dot(final_b, out_wp.astype(F32))

    env = _inv_envelope_2d(T, n_fft, hop, rows)

    def w_spec(a):
        nd = a.ndim
        return pl.BlockSpec(a.shape, lambda bi: (0,) * nd)

    kern = functools.partial(_fused_kernel, T=T, n_layers=n_layers, hop=hop)
    wargs = (ew, embed_b, norm_g, norm_b,
             dww, dwb, w1, b1, w2, b2,
             wm, bm, wp, bp,
             cosb, sinb, env)
    ola = pl.pallas_call(
        kern,
        out_shape=jax.ShapeDtypeStruct((B, rows, hop), F32),
        grid=(B,),
        in_specs=[pl.BlockSpec((1, rows, nm_pad), lambda bi: (bi, 0, 0))]
                 + [w_spec(a) for a in wargs],
        out_specs=pl.BlockSpec((1, rows, hop), lambda bi: (bi, 0, 0)),
        scratch_shapes=[pltpu.VMEM((TOP + T + 8, dim), F32)],
        compiler_params=pltpu.CompilerParams(
            dimension_semantics=("parallel",),
            vmem_limit_bytes=100 * 1024 * 1024,
        ),
    )(x, *wargs)
    start_row = (n_fft // 2) // hop
    return ola[:, start_row:start_row + T - 1, :].reshape(B, (T - 1) * hop)


# speedup vs baseline: 1.0032x; 1.0032x over previous
import functools
import numpy as np

import jax
import jax.numpy as jnp
from jax import lax
from jax.experimental import pallas as pl
from jax.experimental.pallas import tpu as pltpu

F32 = jnp.float32
BF16 = jnp.bfloat16
LN_EPS = 1e-6
KSIZE = 7
HALO = 3
PAD_ROWS = 8
TOP = 8
_GELU_C = 0.7978845608028654


def _layer_norm(x, g, b):
    mu = jnp.mean(x, axis=-1, keepdims=True)
    var = jnp.mean((x - mu) ** 2, axis=-1, keepdims=True)
    return (x - mu) * lax.rsqrt(var + LN_EPS) * g + b


def _normalize(x):
    mu = jnp.mean(x, axis=-1, keepdims=True)
    var = jnp.mean((x - mu) ** 2, axis=-1, keepdims=True)
    return (x - mu) * lax.rsqrt(var + LN_EPS)


def _gelu_tanh(x):
    return 0.5 * x * (1.0 + jnp.tanh(_GELU_C * (x + 0.044715 * (x * x * x))))


def _fused_kernel(x_ref, ew_ref, eb_ref, ng_ref, nb_ref,
                  dww_ref, dwb_ref,
                  w1_ref, b1_ref, w2_ref, b2_ref,
                  wm_ref, bm_ref, wp_ref, bp_ref,
                  cosb_ref, sinb_ref, env_ref, o_ref,
                  xpad_ref, *, T, n_layers, hop):
    srows, dim = xpad_ref.shape
    xpad_ref[0:TOP, :] = jnp.zeros((TOP, dim), F32)
    xpad_ref[TOP + T:srows, :] = jnp.zeros((srows - TOP - T, dim), F32)

    acc = eb_ref[...]
    for j in range(KSIZE):
        acc = acc + jnp.dot(x_ref[0, j:j + T, :], ew_ref[j],
                            preferred_element_type=F32)
    xpad_ref[TOP:TOP + T, :] = _layer_norm(acc, ng_ref[...], nb_ref[...])

    for l in range(n_layers):
        y = dwb_ref[l]
        for j in range(KSIZE):
            y = y + xpad_ref[TOP - HALO + j:TOP - HALO + j + T, :] * dww_ref[l, j:j + 1, :]
        y = _normalize(y)
        h = jnp.dot(y, w1_ref[l], preferred_element_type=F32) + b1_ref[l]
        h = _gelu_tanh(h)
        h = jnp.dot(h, w2_ref[l], preferred_element_type=F32) + b2_ref[l]
        xpad_ref[TOP:TOP + T, :] += h

    xn = _normalize(xpad_ref[TOP:TOP + T, :])
    sm = jnp.dot(xn, wm_ref[...], preferred_element_type=F32) + bm_ref[...]
    sp = jnp.dot(xn, wp_ref[...], preferred_element_type=F32) + bp_ref[...]
    mag = jnp.minimum(jnp.exp(sm), 100.0)
    real = mag * jnp.cos(sp)
    imag = mag * jnp.sin(sp)
    frames = (jnp.dot(real, cosb_ref[...], preferred_element_type=F32)
              + jnp.dot(imag, sinb_ref[...], preferred_element_type=F32))

    r = frames.shape[1] // hop
    orows = o_ref.shape[1]
    ola = None
    for c in range(r):
        part = jnp.pad(frames[:, c * hop:(c + 1) * hop],
                       ((c, orows - T - c), (0, 0)))
        ola = part if ola is None else ola + part
    o_ref[0] = ola * env_ref[...]


def _inv_envelope_2d(t_valid, n_fft, hop, rows):
    n = np.arange(n_fft)
    w2 = (0.5 - 0.5 * np.cos(2.0 * np.pi * n / n_fft)) ** 2
    total = n_fft + (t_valid - 1) * hop
    env = np.zeros((total,), np.float64)
    for t in range(t_valid):
        env[t * hop:t * hop + n_fft] += w2
    start = n_fft // 2
    length = (t_valid - 1) * hop
    inv = 1.0 / np.maximum(env[start:start + length], 1e-8)
    full = np.zeros((rows * hop,), np.float64)
    full[start:start + length] = inv
    return jnp.asarray(full.reshape(rows, hop), F32)


def kernel(mel, embed_w, embed_b, norm_g, norm_b, final_g, final_b, out_wm, out_wp, out_bm, out_bp, cosb, sinb, dww_0, dwb_0, g_0, b_0, w1_0, b1_0, w2_0, b2_0, gamma_0, dww_1, dwb_1, g_1, b_1, w1_1, b1_1, w2_1, b2_1, gamma_1, dww_2, dwb_2, g_2, b_2, w1_2, b1_2, w2_2, b2_2, gamma_2, dww_3, dwb_3, g_3, b_3, w1_3, b1_3, w2_3, b2_3, gamma_3, dww_4, dwb_4, g_4, b_4, w1_4, b1_4, w2_4, b2_4, gamma_4, dww_5, dwb_5, g_5, b_5, w1_5, b1_5, w2_5, b2_5, gamma_5, dww_6, dwb_6, g_6, b_6, w1_6, b1_6, w2_6, b2_6, gamma_6, dww_7, dwb_7, g_7, b_7, w1_7, b1_7, w2_7, b2_7, gamma_7):
    B, n_mels, T = mel.shape
    dim = embed_w.shape[2]
    n_fft = cosb.shape[1]
    hop = n_fft // 4
    n_layers = 8
    rows = T + PAD_ROWS
    nm_pad = 128

    x = jnp.transpose(mel, (0, 2, 1))
    x = jnp.pad(x, ((0, 0), (HALO, PAD_ROWS - HALO), (0, nm_pad - n_mels)))
    ew = jnp.pad(embed_w, ((0, 0), (0, nm_pad - n_mels), (0, 0)))

    dww = jnp.stack([dww_0, dww_1, dww_2, dww_3, dww_4, dww_5, dww_6, dww_7])
    dwb = jnp.stack([dwb_0, dwb_1, dwb_2, dwb_3, dwb_4, dwb_5, dwb_6, dwb_7])

    gs = [g_0, g_1, g_2, g_3, g_4, g_5, g_6, g_7]
    bs = [b_0, b_1, b_2, b_3, b_4, b_5, b_6, b_7]
    w1s = [w1_0, w1_1, w1_2, w1_3, w1_4, w1_5, w1_6, w1_7]
    b1s = [b1_0, b1_1, b1_2, b1_3, b1_4, b1_5, b1_6, b1_7]
    w2s = [w2_0, w2_1, w2_2, w2_3, w2_4, w2_5, w2_6, w2_7]
    b2s = [b2_0, b2_1, b2_2, b2_3, b2_4, b2_5, b2_6, b2_7]
    gms = [gamma_0, gamma_1, gamma_2, gamma_3, gamma_4, gamma_5, gamma_6, gamma_7]
    w1 = jnp.stack([(gs[l][0, :, None] * w1s[l].astype(F32)).astype(BF16)
                    for l in range(n_layers)])
    b1 = jnp.stack([b1s[l] + jnp.dot(bs[l], w1s[l].astype(F32))
                    for l in range(n_layers)])
    w2 = jnp.stack([(w2s[l].astype(F32) * gms[l][0, None, :]).astype(BF16)
                    for l in range(n_layers)])
    b2 = jnp.stack([gms[l] * b2s[l] for l in range(n_layers)])

    wm = (final_g[0, :, None] * out_wm.astype(F32)).astype(BF16)
    bm = out_bm + jnp.dot(final_b, out_wm.astype(F32))
    wp = (final_g[0, :, None] * out_wp.astype(F32)).astype(BF16)
    bp = out_bp + jnp.dot(final_b, out_wp.astype(F32))

    env = _inv_envelope_2d(T, n_fft, hop, rows)

    def w_spec(a):
        nd = a.ndim
        return pl.BlockSpec(a.shape, lambda bi: (0,) * nd)

    kern = functools.partial(_fused_kernel, T=T, n_layers=n_layers, hop=hop)
    wargs = (ew, embed_b, norm_g, norm_b,
             dww, dwb, w1, b1, w2, b2,
             wm, bm, wp, bp,
             cosb, sinb, env)
    ola = pl.pallas_call(
        kern,
        out_shape=jax.ShapeDtypeStruct((B, rows, hop), F32),
        grid=(B,),
        in_specs=[pl.BlockSpec((1, rows, nm_pad), lambda bi: (bi, 0, 0))]
                 + [w_spec(a) for a in wargs],
        out_specs=pl.BlockSpec((1, rows, hop), lambda bi: (bi, 0, 0)),
        scratch_shapes=[pltpu.VMEM((TOP + T + 8, dim), F32)],
        compiler_params=pltpu.CompilerParams(
            dimension_semantics=("parallel",),
            vmem_limit_bytes=100 * 1024 * 1024,
        ),
    )(x, *wargs)
    start_row = (n_fft // 2) // hop
    return ola[:, start_row:start_row + T - 1, :].reshape(B, (T - 1) * hop)
